# baseline (speedup 1.0000x reference)
import functools

import jax
import jax.numpy as jnp
from jax import lax
from jax.experimental import pallas as pl
from jax.experimental.pallas import tpu as pltpu
from jax.experimental.pallas import tpu_sc as plsc

_NUM_CORES = 2
_NUM_SUBCORES = 16
_NUM_WORKERS = _NUM_CORES * _NUM_SUBCORES
_LANES = 16

_D = 128
_CHUNK = 128


def _mf_bpr_body(b_per_w, n_chunk, u_hbm, i_hbm, j_hbm, p_hbm, q_hbm,
                 pi_hbm, pj_hbm, u_v, i_v, j_v, pu_v, qi_v, qj_v,
                 oi_v, oj_v, sem):
    wid = lax.axis_index("s") * _NUM_CORES + lax.axis_index("c")
    base = wid * b_per_w

    pltpu.sync_copy(u_hbm.at[pl.ds(base, b_per_w)], u_v)
    pltpu.sync_copy(i_hbm.at[pl.ds(base, b_per_w)], i_v)
    pltpu.sync_copy(j_hbm.at[pl.ds(base, b_per_w)], j_v)

    for c in range(n_chunk):
        cp_p = pltpu.make_async_copy(
            p_hbm.at[u_v.at[pl.ds(c * _CHUNK, _CHUNK)]], pu_v, sem)
        cp_i = pltpu.make_async_copy(
            q_hbm.at[i_v.at[pl.ds(c * _CHUNK, _CHUNK)]], qi_v, sem)
        cp_j = pltpu.make_async_copy(
            q_hbm.at[j_v.at[pl.ds(c * _CHUNK, _CHUNK)]], qj_v, sem)
        cp_p.start()
        cp_i.start()
        cp_j.start()
        cp_p.wait()
        cp_i.wait()
        cp_j.wait()

        for g in range(_CHUNK // _LANES):
            rows = g * _LANES + lax.iota(jnp.int32, _LANES)

            def body(f, carry, rows=rows):
                acc_i, acc_j = carry
                cols = jnp.full((_LANES,), 0, jnp.int32) + f
                p = plsc.load_gather(pu_v, [rows, cols])
                qi = plsc.load_gather(qi_v, [rows, cols])
                qj = plsc.load_gather(qj_v, [rows, cols])
                return acc_i + p * qi, acc_j + p * qj

            zero = jnp.zeros((_LANES,), jnp.float32)
            acc_i, acc_j = lax.fori_loop(0, _D, body, (zero, zero), unroll=4)
            off = c * _CHUNK + g * _LANES
            oi_v[pl.ds(off, _LANES)] = acc_i
            oj_v[pl.ds(off, _LANES)] = acc_j

    pltpu.sync_copy(oi_v, pi_hbm.at[pl.ds(base, b_per_w)])
    pltpu.sync_copy(oj_v, pj_hbm.at[pl.ds(base, b_per_w)])


@jax.jit
def kernel(u, i, j, P, Q):
    batch = u.shape[0]
    b_per_w = batch // _NUM_WORKERS
    n_chunk = b_per_w // _CHUNK

    mesh = plsc.VectorSubcoreMesh(
        core_axis_name="c", subcore_axis_name="s",
        num_cores=_NUM_CORES, num_subcores=_NUM_SUBCORES)

    run = pl.kernel(
        functools.partial(_mf_bpr_body, b_per_w, n_chunk),
        out_type=(
            jax.ShapeDtypeStruct((batch,), jnp.float32),
            jax.ShapeDtypeStruct((batch,), jnp.float32),
        ),
        mesh=mesh,
        compiler_params=pltpu.CompilerParams(needs_layout_passes=False),
        scratch_types=[
            pltpu.VMEM((b_per_w,), jnp.int32),
            pltpu.VMEM((b_per_w,), jnp.int32),
            pltpu.VMEM((b_per_w,), jnp.int32),
            pltpu.VMEM((_CHUNK, _D), jnp.float32),
            pltpu.VMEM((_CHUNK, _D), jnp.float32),
            pltpu.VMEM((_CHUNK, _D), jnp.float32),
            pltpu.VMEM((b_per_w,), jnp.float32),
            pltpu.VMEM((b_per_w,), jnp.float32),
            pltpu.SemaphoreType.DMA,
        ],
    )
    pi, pj = run(u.astype(jnp.int32), i.astype(jnp.int32), j.astype(jnp.int32),
                 P, Q)
    return pi, pj


# speedup vs baseline: 3.3382x; 3.3382x over previous
import functools

import jax
import jax.numpy as jnp
from jax import lax
from jax.experimental import pallas as pl
from jax.experimental.pallas import tpu as pltpu
from jax.experimental.pallas import tpu_sc as plsc

_NUM_CORES = 2
_NUM_SUBCORES = 16
_NUM_WORKERS = _NUM_CORES * _NUM_SUBCORES
_LANES = 16

_D = 128
_CHUNK = 128
_NBUF = 2


def _mf_bpr_body(b_per_w, n_chunk, u_hbm, i_hbm, j_hbm, p_hbm, q_hbm,
                 pi_hbm, pj_hbm, u_v, i_v, j_v, pu_v, qi_v, qj_v,
                 oi_v, oj_v, sems):
    wid = lax.axis_index("s") * _NUM_CORES + lax.axis_index("c")
    base = wid * b_per_w

    pltpu.sync_copy(u_hbm.at[pl.ds(base, b_per_w)], u_v)
    pltpu.sync_copy(i_hbm.at[pl.ds(base, b_per_w)], i_v)
    pltpu.sync_copy(j_hbm.at[pl.ds(base, b_per_w)], j_v)

    lane = lax.iota(jnp.int32, _LANES)
    zero = jnp.zeros((_LANES,), jnp.float32)

    def start(c):
        par = c % _NBUF
        sl = pl.ds(c * _CHUNK, _CHUNK)
        cps = (
            pltpu.make_async_copy(p_hbm.at[u_v.at[sl]], pu_v.at[par], sems.at[par]),
            pltpu.make_async_copy(q_hbm.at[i_v.at[sl]], qi_v.at[par], sems.at[par]),
            pltpu.make_async_copy(q_hbm.at[j_v.at[sl]], qj_v.at[par], sems.at[par]),
        )
        for cp in cps:
            cp.start()
        return cps

    inflight = {0: start(0)}
    for c in range(n_chunk):
        if c + 1 < n_chunk:
            inflight[c + 1] = start(c + 1)
        for cp in inflight.pop(c):
            cp.wait()
        par = c % _NBUF
        pu, qi, qj = pu_v.at[par], qi_v.at[par], qj_v.at[par]

        for g in range(_CHUNK // _LANES):
            rows = g * _LANES + lane

            def body(k, carry, pu=pu, qi=qi, qj=qj, rows=rows):
                t0, ai0, ai1, aj0, aj1 = carry
                p0 = plsc.load_gather(pu, [rows, t0])
                a0 = plsc.load_gather(qi, [rows, t0])
                b0 = plsc.load_gather(qj, [rows, t0])
                t1 = (t0 + 1) & (_D - 1)
                p1 = plsc.load_gather(pu, [rows, t1])
                a1 = plsc.load_gather(qi, [rows, t1])
                b1 = plsc.load_gather(qj, [rows, t1])
                t2 = (t1 + 1) & (_D - 1)
                return (t2, ai0 + p0 * a0, ai1 + p1 * a1,
                        aj0 + p0 * b0, aj1 + p1 * b1)

            init = (lane, zero, zero, zero, zero)
            _, ai0, ai1, aj0, aj1 = lax.fori_loop(0, _D // 2, body, init,
                                                  unroll=2)
            off = c * _CHUNK + g * _LANES
            oi_v[pl.ds(off, _LANES)] = ai0 + ai1
            oj_v[pl.ds(off, _LANES)] = aj0 + aj1

    pltpu.sync_copy(oi_v, pi_hbm.at[pl.ds(base, b_per_w)])
    pltpu.sync_copy(oj_v, pj_hbm.at[pl.ds(base, b_per_w)])


@jax.jit
def kernel(u, i, j, P, Q):
    batch = u.shape[0]
    b_per_w = batch // _NUM_WORKERS
    n_chunk = b_per_w // _CHUNK

    mesh = plsc.VectorSubcoreMesh(
        core_axis_name="c", subcore_axis_name="s",
        num_cores=_NUM_CORES, num_subcores=_NUM_SUBCORES)

    run = pl.kernel(
        functools.partial(_mf_bpr_body, b_per_w, n_chunk),
        out_type=(
            jax.ShapeDtypeStruct((batch,), jnp.float32),
            jax.ShapeDtypeStruct((batch,), jnp.float32),
        ),
        mesh=mesh,
        compiler_params=pltpu.CompilerParams(needs_layout_passes=False),
        scratch_types=[
            pltpu.VMEM((b_per_w,), jnp.int32),
            pltpu.VMEM((b_per_w,), jnp.int32),
            pltpu.VMEM((b_per_w,), jnp.int32),
            pltpu.VMEM((_NBUF, _CHUNK, _D), jnp.float32),
            pltpu.VMEM((_NBUF, _CHUNK, _D), jnp.float32),
            pltpu.VMEM((_NBUF, _CHUNK, _D), jnp.float32),
            pltpu.VMEM((b_per_w,), jnp.float32),
            pltpu.VMEM((b_per_w,), jnp.float32),
            pltpu.SemaphoreType.DMA((_NBUF,)),
        ],
    )
    pi, pj = run(u.astype(jnp.int32), i.astype(jnp.int32), j.astype(jnp.int32),
                 P, Q)
    return pi, pj


# speedup vs baseline: 3.4379x; 1.0299x over previous
import functools

import jax
import jax.numpy as jnp
from jax import lax
from jax.experimental import pallas as pl
from jax.experimental.pallas import tpu as pltpu
from jax.experimental.pallas import tpu_sc as plsc

_NUM_CORES = 2
_NUM_SUBCORES = 16
_NUM_WORKERS = _NUM_CORES * _NUM_SUBCORES
_LANES = 16

_D = 128
_CHUNK = 64
_NBUF = 4


def _mf_bpr_body(b_per_w, n_chunk, u_hbm, i_hbm, j_hbm, p_hbm, q_hbm,
                 pi_hbm, pj_hbm, u_v, i_v, j_v, pu_v, qi_v, qj_v,
                 oi_v, oj_v, sems):
    wid = lax.axis_index("s") * _NUM_CORES + lax.axis_index("c")
    base = wid * b_per_w

    pltpu.sync_copy(u_hbm.at[pl.ds(base, b_per_w)], u_v)
    pltpu.sync_copy(i_hbm.at[pl.ds(base, b_per_w)], i_v)
    pltpu.sync_copy(j_hbm.at[pl.ds(base, b_per_w)], j_v)

    lane = lax.iota(jnp.int32, _LANES)
    zero = jnp.zeros((_LANES,), jnp.float32)

    def start(c):
        par = c % _NBUF
        sl = pl.ds(c * _CHUNK, _CHUNK)
        cps = (
            pltpu.make_async_copy(p_hbm.at[u_v.at[sl]], pu_v.at[par], sems.at[par]),
            pltpu.make_async_copy(q_hbm.at[i_v.at[sl]], qi_v.at[par], sems.at[par]),
            pltpu.make_async_copy(q_hbm.at[j_v.at[sl]], qj_v.at[par], sems.at[par]),
        )
        for cp in cps:
            cp.start()
        return cps

    inflight = {c: start(c) for c in range(min(_NBUF - 1, n_chunk))}
    for c in range(n_chunk):
        if c + _NBUF - 1 < n_chunk:
            inflight[c + _NBUF - 1] = start(c + _NBUF - 1)
        for cp in inflight.pop(c):
            cp.wait()
        par = c % _NBUF
        pu, qi, qj = pu_v.at[par], qi_v.at[par], qj_v.at[par]

        for g in range(_CHUNK // _LANES):
            rows = g * _LANES + lane

            def body(k, carry, pu=pu, qi=qi, qj=qj, rows=rows):
                t0, ai0, ai1, aj0, aj1 = carry
                p0 = plsc.load_gather(pu, [rows, t0])
                a0 = plsc.load_gather(qi, [rows, t0])
                b0 = plsc.load_gather(qj, [rows, t0])
                t1 = (t0 + 1) & (_D - 1)
                p1 = plsc.load_gather(pu, [rows, t1])
                a1 = plsc.load_gather(qi, [rows, t1])
                b1 = plsc.load_gather(qj, [rows, t1])
                t2 = (t1 + 1) & (_D - 1)
                return (t2, ai0 + p0 * a0, ai1 + p1 * a1,
                        aj0 + p0 * b0, aj1 + p1 * b1)

            init = (lane, zero, zero, zero, zero)
            _, ai0, ai1, aj0, aj1 = lax.fori_loop(0, _D // 2, body, init,
                                                  unroll=2)
            off = c * _CHUNK + g * _LANES
            oi_v[pl.ds(off, _LANES)] = ai0 + ai1
            oj_v[pl.ds(off, _LANES)] = aj0 + aj1

    pltpu.sync_copy(oi_v, pi_hbm.at[pl.ds(base, b_per_w)])
    pltpu.sync_copy(oj_v, pj_hbm.at[pl.ds(base, b_per_w)])


@jax.jit
def kernel(u, i, j, P, Q):
    batch = u.shape[0]
    b_per_w = batch // _NUM_WORKERS
    n_chunk = b_per_w // _CHUNK

    mesh = plsc.VectorSubcoreMesh(
        core_axis_name="c", subcore_axis_name="s",
        num_cores=_NUM_CORES, num_subcores=_NUM_SUBCORES)

    run = pl.kernel(
        functools.partial(_mf_bpr_body, b_per_w, n_chunk),
        out_type=(
            jax.ShapeDtypeStruct((batch,), jnp.float32),
            jax.ShapeDtypeStruct((batch,), jnp.float32),
        ),
        mesh=mesh,
        compiler_params=pltpu.CompilerParams(needs_layout_passes=False),
        scratch_types=[
            pltpu.VMEM((b_per_w,), jnp.int32),
            pltpu.VMEM((b_per_w,), jnp.int32),
            pltpu.VMEM((b_per_w,), jnp.int32),
            pltpu.VMEM((_NBUF, _CHUNK, _D), jnp.float32),
            pltpu.VMEM((_NBUF, _CHUNK, _D), jnp.float32),
            pltpu.VMEM((_NBUF, _CHUNK, _D), jnp.float32),
            pltpu.VMEM((b_per_w,), jnp.float32),
            pltpu.VMEM((b_per_w,), jnp.float32),
            pltpu.SemaphoreType.DMA((_NBUF,)),
        ],
    )
    pi, pj = run(u.astype(jnp.int32), i.astype(jnp.int32), j.astype(jnp.int32),
                 P, Q)
    return pi, pj


# speedup vs baseline: 3.7948x; 1.1038x over previous
import functools

import jax
import jax.numpy as jnp
from jax import lax
from jax.experimental import pallas as pl
from jax.experimental.pallas import tpu as pltpu
from jax.experimental.pallas import tpu_sc as plsc

_NUM_CORES = 2
_NUM_SUBCORES = 16
_NUM_WORKERS = _NUM_CORES * _NUM_SUBCORES
_LANES = 16

_D = 128
_CHUNK = 64
_NBUF = 4


def _mf_bpr_body(b_per_w, n_chunk, u_hbm, i_hbm, j_hbm, p_hbm, q_hbm,
                 pi_hbm, pj_hbm, u_v, i_v, j_v, pu_v, qi_v, qj_v,
                 oi_v, oj_v, sems, osem):
    wid = lax.axis_index("s") * _NUM_CORES + lax.axis_index("c")
    base = wid * b_per_w

    stage = (
        pltpu.make_async_copy(u_hbm.at[pl.ds(base, b_per_w)], u_v, osem),
        pltpu.make_async_copy(i_hbm.at[pl.ds(base, b_per_w)], i_v, osem),
        pltpu.make_async_copy(j_hbm.at[pl.ds(base, b_per_w)], j_v, osem),
    )
    for cp in stage:
        cp.start()
    for cp in stage:
        cp.wait()

    lane = lax.iota(jnp.int32, _LANES)
    zero = jnp.zeros((_LANES,), jnp.float32)

    def copies(c, par):
        sl = pl.ds(c * _CHUNK, _CHUNK)
        return (
            pltpu.make_async_copy(p_hbm.at[u_v.at[sl]], pu_v.at[par], sems.at[par]),
            pltpu.make_async_copy(q_hbm.at[i_v.at[sl]], qi_v.at[par], sems.at[par]),
            pltpu.make_async_copy(q_hbm.at[j_v.at[sl]], qj_v.at[par], sems.at[par]),
        )

    def start(c, par):
        for cp in copies(c, par):
            cp.start()

    def process(c, par):
        for cp in copies(c, par):
            cp.wait()
        pu, qi, qj = pu_v.at[par], qi_v.at[par], qj_v.at[par]
        out_base = c * _CHUNK

        def group(g, _):
            rows = g * _LANES + lane

            def body(k, carry):
                t0, ai0, ai1, aj0, aj1 = carry
                p0 = plsc.load_gather(pu, [rows, t0])
                a0 = plsc.load_gather(qi, [rows, t0])
                b0 = plsc.load_gather(qj, [rows, t0])
                t1 = (t0 + 1) & (_D - 1)
                p1 = plsc.load_gather(pu, [rows, t1])
                a1 = plsc.load_gather(qi, [rows, t1])
                b1 = plsc.load_gather(qj, [rows, t1])
                t2 = (t1 + 1) & (_D - 1)
                return (t2, ai0 + p0 * a0, ai1 + p1 * a1,
                        aj0 + p0 * b0, aj1 + p1 * b1)

            init = (lane, zero, zero, zero, zero)
            _, ai0, ai1, aj0, aj1 = lax.fori_loop(0, _D // 2, body, init,
                                                  unroll=2)
            off = out_base + g * _LANES
            oi_v[pl.ds(off, _LANES)] = ai0 + ai1
            oj_v[pl.ds(off, _LANES)] = aj0 + aj1
            return 0

        lax.fori_loop(0, _CHUNK // _LANES, group, 0)

    for b in range(_NBUF - 1):
        start(b, b)

    def super_step(s, _):
        for b in range(_NBUF):
            c = s * _NBUF + b
            pre = c + _NBUF - 1

            @pl.when(pre < n_chunk)
            def _():
                start(pre, pre % _NBUF)

            process(c, b)
        return 0

    lax.fori_loop(0, n_chunk // _NBUF, super_step, 0)

    out = (
        pltpu.make_async_copy(oi_v, pi_hbm.at[pl.ds(base, b_per_w)], osem),
        pltpu.make_async_copy(oj_v, pj_hbm.at[pl.ds(base, b_per_w)], osem),
    )
    for cp in out:
        cp.start()
    for cp in out:
        cp.wait()


@jax.jit
def kernel(u, i, j, P, Q):
    batch = u.shape[0]
    b_per_w = batch // _NUM_WORKERS
    n_chunk = b_per_w // _CHUNK

    mesh = plsc.VectorSubcoreMesh(
        core_axis_name="c", subcore_axis_name="s",
        num_cores=_NUM_CORES, num_subcores=_NUM_SUBCORES)

    run = pl.kernel(
        functools.partial(_mf_bpr_body, b_per_w, n_chunk),
        out_type=(
            jax.ShapeDtypeStruct((batch,), jnp.float32),
            jax.ShapeDtypeStruct((batch,), jnp.float32),
        ),
        mesh=mesh,
        compiler_params=pltpu.CompilerParams(needs_layout_passes=False),
        scratch_types=[
            pltpu.VMEM((b_per_w,), jnp.int32),
            pltpu.VMEM((b_per_w,), jnp.int32),
            pltpu.VMEM((b_per_w,), jnp.int32),
            pltpu.VMEM((_NBUF, _CHUNK, _D), jnp.float32),
            pltpu.VMEM((_NBUF, _CHUNK, _D), jnp.float32),
            pltpu.VMEM((_NBUF, _CHUNK, _D), jnp.float32),
            pltpu.VMEM((b_per_w,), jnp.float32),
            pltpu.VMEM((b_per_w,), jnp.float32),
            pltpu.SemaphoreType.DMA((_NBUF,)),
            pltpu.SemaphoreType.DMA,
        ],
    )
    pi, pj = run(u.astype(jnp.int32), i.astype(jnp.int32), j.astype(jnp.int32),
                 P, Q)
    return pi, pj


# speedup vs baseline: 3.8281x; 1.0088x over previous
import functools

import jax
import jax.numpy as jnp
from jax import lax
from jax.experimental import pallas as pl
from jax.experimental.pallas import tpu as pltpu
from jax.experimental.pallas import tpu_sc as plsc

_NUM_CORES = 2
_NUM_SUBCORES = 16
_NUM_WORKERS = _NUM_CORES * _NUM_SUBCORES
_LANES = 16

_D = 128
_CHUNK = 32
_NBUF = 8


def _mf_bpr_body(b_per_w, n_chunk, u_hbm, i_hbm, j_hbm, p_hbm, q_hbm,
                 pi_hbm, pj_hbm, u_v, i_v, j_v, pu_v, qi_v, qj_v,
                 oi_v, oj_v, sems, osem):
    wid = lax.axis_index("s") * _NUM_CORES + lax.axis_index("c")
    base = wid * b_per_w

    stage = (
        pltpu.make_async_copy(u_hbm.at[pl.ds(base, b_per_w)], u_v, osem),
        pltpu.make_async_copy(i_hbm.at[pl.ds(base, b_per_w)], i_v, osem),
        pltpu.make_async_copy(j_hbm.at[pl.ds(base, b_per_w)], j_v, osem),
    )
    for cp in stage:
        cp.start()
    for cp in stage:
        cp.wait()

    lane = lax.iota(jnp.int32, _LANES)
    zero = jnp.zeros((_LANES,), jnp.float32)

    def copies(c, par):
        sl = pl.ds(c * _CHUNK, _CHUNK)
        return (
            pltpu.make_async_copy(p_hbm.at[u_v.at[sl]], pu_v.at[par], sems.at[par]),
            pltpu.make_async_copy(q_hbm.at[i_v.at[sl]], qi_v.at[par], sems.at[par]),
            pltpu.make_async_copy(q_hbm.at[j_v.at[sl]], qj_v.at[par], sems.at[par]),
        )

    def start(c, par):
        for cp in copies(c, par):
            cp.start()

    def process(c, par):
        for cp in copies(c, par):
            cp.wait()
        pu, qi, qj = pu_v.at[par], qi_v.at[par], qj_v.at[par]
        out_base = c * _CHUNK

        def group(g, _):
            rows = g * _LANES + lane

            def body(k, carry):
                t0, ai0, ai1, aj0, aj1 = carry
                p0 = plsc.load_gather(pu, [rows, t0])
                a0 = plsc.load_gather(qi, [rows, t0])
                b0 = plsc.load_gather(qj, [rows, t0])
                t1 = (t0 + 1) & (_D - 1)
                p1 = plsc.load_gather(pu, [rows, t1])
                a1 = plsc.load_gather(qi, [rows, t1])
                b1 = plsc.load_gather(qj, [rows, t1])
                t2 = (t1 + 1) & (_D - 1)
                return (t2, ai0 + p0 * a0, ai1 + p1 * a1,
                        aj0 + p0 * b0, aj1 + p1 * b1)

            init = (lane, zero, zero, zero, zero)
            _, ai0, ai1, aj0, aj1 = lax.fori_loop(0, _D // 2, body, init,
                                                  unroll=2)
            off = out_base + g * _LANES
            oi_v[pl.ds(off, _LANES)] = ai0 + ai1
            oj_v[pl.ds(off, _LANES)] = aj0 + aj1
            return 0

        lax.fori_loop(0, _CHUNK // _LANES, group, 0)

    for b in range(_NBUF - 1):
        start(b, b)

    def super_step(s, _):
        for b in range(_NBUF):
            c = s * _NBUF + b
            pre = c + _NBUF - 1

            @pl.when(pre < n_chunk)
            def _():
                start(pre, pre % _NBUF)

            process(c, b)
        return 0

    lax.fori_loop(0, n_chunk // _NBUF, super_step, 0)

    out = (
        pltpu.make_async_copy(oi_v, pi_hbm.at[pl.ds(base, b_per_w)], osem),
        pltpu.make_async_copy(oj_v, pj_hbm.at[pl.ds(base, b_per_w)], osem),
    )
    for cp in out:
        cp.start()
    for cp in out:
        cp.wait()


@jax.jit
def kernel(u, i, j, P, Q):
    batch = u.shape[0]
    b_per_w = batch // _NUM_WORKERS
    n_chunk = b_per_w // _CHUNK

    mesh = plsc.VectorSubcoreMesh(
        core_axis_name="c", subcore_axis_name="s",
        num_cores=_NUM_CORES, num_subcores=_NUM_SUBCORES)

    run = pl.kernel(
        functools.partial(_mf_bpr_body, b_per_w, n_chunk),
        out_type=(
            jax.ShapeDtypeStruct((batch,), jnp.float32),
            jax.ShapeDtypeStruct((batch,), jnp.float32),
        ),
        mesh=mesh,
        compiler_params=pltpu.CompilerParams(needs_layout_passes=False),
        scratch_types=[
            pltpu.VMEM((b_per_w,), jnp.int32),
            pltpu.VMEM((b_per_w,), jnp.int32),
            pltpu.VMEM((b_per_w,), jnp.int32),
            pltpu.VMEM((_NBUF, _CHUNK, _D), jnp.float32),
            pltpu.VMEM((_NBUF, _CHUNK, _D), jnp.float32),
            pltpu.VMEM((_NBUF, _CHUNK, _D), jnp.float32),
            pltpu.VMEM((b_per_w,), jnp.float32),
            pltpu.VMEM((b_per_w,), jnp.float32),
            pltpu.SemaphoreType.DMA((_NBUF,)),
            pltpu.SemaphoreType.DMA,
        ],
    )
    pi, pj = run(u.astype(jnp.int32), i.astype(jnp.int32), j.astype(jnp.int32),
                 P, Q)
    return pi, pj
